# baseline (speedup 1.0000x reference)
import functools

import jax
import jax.numpy as jnp
from jax import lax
from jax.experimental import pallas as pl
from jax.experimental.pallas import tpu as pltpu

BATCH = 16384
HIDDEN = 64
GAMMA = 12.0

BLK = 2048
NBLK = BATCH // BLK
NDUP = 2
NQ = 2 * NDUP
QROWS = BLK // NQ


def _body(idx_h, idx_r, idx_t, ent_a, ent_b, rel_a, rel_b, out_ref, *rest):
    bufs = rest[:3 * NQ]
    sems = rest[3 * NQ]
    k = pl.program_id(0)

    idxs = (idx_h, idx_r, idx_t)
    tabs = ((ent_a, ent_b), (rel_a, rel_b), (ent_a, ent_b))

    def issue_block(blk, par):
        def enqueue(r, carry):
            for t in range(3):
                for q in range(NQ):
                    i = idxs[t][blk * BLK + q * QROWS + r]
                    pltpu.async_copy(
                        tabs[t][q % NDUP].at[i],
                        bufs[t * NQ + q].at[par, r],
                        sems.at[par, t * NQ + q])
            return carry

        lax.fori_loop(0, QROWS, enqueue, 0, unroll=2)

    par = lax.rem(k, 2)
    nxt = lax.rem(k + 1, 2)

    @pl.when(k == 0)
    def _():
        issue_block(0, 0)

    @pl.when(k + 1 < NBLK)
    def _():
        issue_block(k + 1, nxt)

    for tq in range(3 * NQ):
        pltpu.make_async_copy(
            ent_a.at[pl.ds(0, QROWS)], bufs[tq].at[par],
            sems.at[par, tq]).wait()

    h = jnp.concatenate([bufs[q][par] for q in range(NQ)], axis=0)
    r = jnp.concatenate([bufs[NQ + q][par] for q in range(NQ)], axis=0)
    t = jnp.concatenate([bufs[2 * NQ + q][par] for q in range(NQ)], axis=0)
    d = jnp.abs(h + r - t)
    out_ref[...] = GAMMA - jnp.sum(d, axis=1, keepdims=True)


@jax.jit
def _score(heads, rels, tails, entity_embedding, relation_embedding):
    grid_spec = pltpu.PrefetchScalarGridSpec(
        num_scalar_prefetch=3,
        grid=(NBLK,),
        in_specs=[pl.BlockSpec(memory_space=pl.ANY)] * 4,
        out_specs=pl.BlockSpec((BLK, 1), lambda k, *p: (k, 0)),
        scratch_shapes=(
            [pltpu.VMEM((2, QROWS, HIDDEN), jnp.float32)
             for _ in range(3 * NQ)]
            + [pltpu.SemaphoreType.DMA((2, 3 * NQ))]),
    )
    fn = pl.pallas_call(
        _body,
        grid_spec=grid_spec,
        out_shape=jax.ShapeDtypeStruct((BATCH, 1), jnp.float32),
        compiler_params=pltpu.CompilerParams(
            dimension_semantics=("arbitrary",)),
    )
    return fn(heads, rels, tails,
              entity_embedding, entity_embedding,
              relation_embedding, relation_embedding)


def kernel(sample, entity_embedding, relation_embedding):
    sample = sample.astype(jnp.int32)
    heads = sample[:, 0]
    rels = sample[:, 1]
    tails = sample[:, 2]
    return _score(heads, rels, tails, entity_embedding, relation_embedding)
